# Initial kernel scaffold; baseline (speedup 1.0000x reference)
import jax
import jax.numpy as jnp
from jax.experimental import pallas as pl


def kernel(mean, var, edge_index, edge_weight0, edge_weight1, W_mean, b_mean, W_var, b_var):
    raise NotImplementedError("write your pallas kernel here")


# speedup vs baseline: 3.3866x; 3.3866x over previous
import functools

import jax
import jax.numpy as jnp
from jax import lax
from jax.experimental import pallas as pl
from jax.experimental.pallas import tpu as pltpu
from jax.experimental.pallas import tpu_sc as plsc

N = 10000
E = 320000
D = 128

NSUB = 16
CH = 128
NCH = E // CH
NP = 10240
RP = NP // NSUB
RZ = 128


_BLK = 2000


def _dense_body(mean_ref, var_ref, wm_ref, bm_ref, wv_ref, bv_ref, out_ref):
    m = jnp.dot(mean_ref[...], wm_ref[...], preferred_element_type=jnp.float32)
    m = m + bm_ref[...]
    m = jnp.where(m > 0, m, jnp.exp(jnp.minimum(m, 0.0)) - 1.0)
    v = jnp.dot(var_ref[...], wv_ref[...], preferred_element_type=jnp.float32)
    v = v + bv_ref[...]
    v = jnp.maximum(v, 0.0) + 1e-6
    att = jnp.exp(-v)
    out_ref[0] = m * att
    out_ref[1] = v * att * att


def _dense(mean, var, wm, bm, wv, bv):
    grid = (N // _BLK,)
    return pl.pallas_call(
        _dense_body,
        grid=grid,
        in_specs=[
            pl.BlockSpec((_BLK, D), lambda i: (i, 0)),
            pl.BlockSpec((_BLK, D), lambda i: (i, 0)),
            pl.BlockSpec((D, D), lambda i: (0, 0)),
            pl.BlockSpec((1, D), lambda i: (0, 0)),
            pl.BlockSpec((D, D), lambda i: (0, 0)),
            pl.BlockSpec((1, D), lambda i: (0, 0)),
        ],
        out_specs=pl.BlockSpec((2, _BLK, D), lambda i: (0, i, 0)),
        out_shape=jax.ShapeDtypeStruct((2, N, D), jnp.float32),
    )(mean, var, wm, bm, wv, bv)


def _agg_body(x_hbm, row_hbm, col_hbm, w_hbm, out_hbm,
              acc, colbuf, rowbuf, wbuf, gbuf, sbuf, sem):
    c = lax.axis_index("c")
    s = lax.axis_index("s")
    cN = c * N
    cE = c * E
    z16 = jnp.zeros((16,), jnp.float32)

    def zbody(r, carry):
        for k in range(D // 16):
            sbuf[r, pl.ds(k * 16, 16)] = z16
        return carry

    lax.fori_loop(0, RZ, zbody, 0)
    r0 = s * RP
    for j in range(RP // RZ):
        pltpu.sync_copy(sbuf, acc.at[pl.ds(r0 + j * RZ, RZ)])
    plsc.subcore_barrier()

    iota = lax.iota(jnp.int32, 16)
    nb = CH // 16

    def cbody(i, carry):
        off = i * CH
        pltpu.sync_copy(col_hbm.at[pl.ds(off, CH)], colbuf)
        pltpu.sync_copy(row_hbm.at[pl.ds(off, CH)], rowbuf)
        pltpu.sync_copy(w_hbm.at[pl.ds(cE + off, CH)], wbuf)
        for k in range(nb):
            colbuf[pl.ds(k * 16, 16)] = colbuf[pl.ds(k * 16, 16)] + cN
        pltpu.async_copy(x_hbm.at[colbuf], gbuf, sem).wait()
        def ebody(e, ecarry):
            w = plsc.load_gather(wbuf, [jnp.full((16,), e, jnp.int32)])
            for k in range(D // 16):
                sl = gbuf[e, pl.ds(k * 16, 16)]
                gbuf[e, pl.ds(k * 16, 16)] = sl * w
            return ecarry

        lax.fori_loop(0, CH, ebody, 0)
        pltpu.sync_copy(gbuf, acc.at[rowbuf], add=True)
        return carry

    lax.fori_loop(s * NCH // NSUB, (s + 1) * NCH // NSUB, cbody, 0)
    plsc.subcore_barrier()

    for j in range(RP // RZ):
        pltpu.sync_copy(acc.at[pl.ds(r0 + j * RZ, RZ)], sbuf)
        pltpu.sync_copy(sbuf, out_hbm.at[pl.ds(c * NP + r0 + j * RZ, RZ)])


def _agg(x_all, row, col, w_all):
    mesh = plsc.VectorSubcoreMesh(core_axis_name="c", subcore_axis_name="s")
    f = functools.partial(
        pl.kernel,
        out_type=jax.ShapeDtypeStruct((2 * NP, D), jnp.float32),
        mesh=mesh,
        compiler_params=pltpu.CompilerParams(needs_layout_passes=False),
        scratch_types=[
            pltpu.VMEM_SHARED((NP, D), jnp.float32),
            pltpu.VMEM((CH,), jnp.int32),
            pltpu.VMEM((CH,), jnp.int32),
            pltpu.VMEM((CH,), jnp.float32),
            pltpu.VMEM((CH, D), jnp.float32),
            pltpu.VMEM((RZ, D), jnp.float32),
            pltpu.SemaphoreType.DMA,
        ],
    )(_agg_body)
    return f(x_all, row, col, w_all)


def kernel(mean, var, edge_index, edge_weight0, edge_weight1,
           W_mean, b_mean, W_var, b_var):
    xs = _dense(mean, var, W_mean, b_mean.reshape(1, D),
                W_var, b_var.reshape(1, D))
    x_all = xs.reshape(2 * N, D)
    row = edge_index[0]
    col = edge_index[1]
    w_all = jnp.concatenate([edge_weight0, edge_weight1])
    out = _agg(x_all, row, col, w_all)
    return out[:N], out[NP:NP + N]


# speedup vs baseline: 6.4380x; 1.9010x over previous
import functools

import jax
import jax.numpy as jnp
from jax import lax
from jax.experimental import pallas as pl
from jax.experimental.pallas import tpu as pltpu
from jax.experimental.pallas import tpu_sc as plsc

N = 10000
E = 320000
D = 128

NSUB = 16
CH = 112
NCHT = 180
EPAD = NSUB * NCHT * CH
NP = 10112
RP = NP // NSUB
NB = 3
NI = 6


_BLK = 2000


def _dense_body(mean_ref, var_ref, wm_ref, bm_ref, wv_ref, bv_ref, out_ref):
    m = jnp.dot(mean_ref[...], wm_ref[...], preferred_element_type=jnp.float32)
    m = m + bm_ref[...]
    m = jnp.where(m > 0, m, jnp.exp(jnp.minimum(m, 0.0)) - 1.0)
    v = jnp.dot(var_ref[...], wv_ref[...], preferred_element_type=jnp.float32)
    v = v + bv_ref[...]
    v = jnp.maximum(v, 0.0) + 1e-6
    att = jnp.exp(-v)
    out_ref[0] = m * att
    out_ref[1] = v * att * att


def _dense(mean, var, wm, bm, wv, bv):
    grid = (N // _BLK,)
    return pl.pallas_call(
        _dense_body,
        grid=grid,
        in_specs=[
            pl.BlockSpec((_BLK, D), lambda i: (i, 0)),
            pl.BlockSpec((_BLK, D), lambda i: (i, 0)),
            pl.BlockSpec((D, D), lambda i: (0, 0)),
            pl.BlockSpec((1, D), lambda i: (0, 0)),
            pl.BlockSpec((D, D), lambda i: (0, 0)),
            pl.BlockSpec((1, D), lambda i: (0, 0)),
        ],
        out_specs=pl.BlockSpec((2, _BLK, D), lambda i: (0, i, 0)),
        out_shape=jax.ShapeDtypeStruct((2, N, D), jnp.float32),
    )(mean, var, wm, bm, wv, bv)


def _agg_body(x_hbm, row_hbm, col_hbm, w_hbm, out_hbm, acc, *scr):
    colbuf = scr[0:NI]
    rowbuf = scr[NI:2 * NI]
    wbuf = scr[2 * NI:3 * NI]
    gbuf = scr[3 * NI:3 * NI + NB]
    o = 3 * NI + NB
    colsem = scr[o:o + NI]
    rowsem = scr[o + NI:o + 2 * NI]
    wsem = scr[o + 2 * NI:o + 3 * NI]
    gsem = scr[o + 3 * NI:o + 3 * NI + NB]
    ssem = scr[o + 3 * NI + NB:o + 3 * NI + 2 * NB]

    c = lax.axis_index("c")
    s = lax.axis_index("s")
    cN = c * N
    cE = c * EPAD
    z16 = jnp.zeros((16,), jnp.float32)
    t0 = s * NCHT
    r0 = s * RP
    g0 = gbuf[0]

    def start_idx(i, e):
        off = (t0 + i) * CH
        pltpu.async_copy(col_hbm.at[pl.ds(off, CH)], colbuf[e], colsem[e])
        pltpu.async_copy(row_hbm.at[pl.ds(off, CH)], rowbuf[e], rowsem[e])
        pltpu.async_copy(w_hbm.at[pl.ds(cE + off, CH)], wbuf[e], wsem[e])

    def start_gather(i, e, b):
        off = (t0 + i) * CH
        pltpu.make_async_copy(col_hbm.at[pl.ds(off, CH)], colbuf[e],
                              colsem[e]).wait()
        for k in range(CH // 16):
            colbuf[e][pl.ds(k * 16, 16)] = colbuf[e][pl.ds(k * 16, 16)] + cN
        pltpu.async_copy(x_hbm.at[colbuf[e]], gbuf[b], gsem[b])

    def wait_scatter(b):
        pltpu.make_async_copy(gbuf[b], acc.at[rowbuf[0]], ssem[b]).wait()

    for i in range(3):
        start_idx(i, i)

    def zbody(r, carry):
        for k in range(D // 16):
            g0[r, pl.ds(k * 16, 16)] = z16
        return carry

    lax.fori_loop(0, CH, zbody, 0)
    for j in range(RP // CH):
        pltpu.sync_copy(g0.at[pl.ds(0, CH)], acc.at[pl.ds(r0 + j * CH, CH)])
    rem = RP - (RP // CH) * CH
    pltpu.sync_copy(g0.at[pl.ds(0, rem)],
                    acc.at[pl.ds(r0 + (RP // CH) * CH, rem)])

    start_gather(0, 0, 0)
    start_gather(1, 1, 1)
    plsc.subcore_barrier()

    def scale(b, e):
        gb = gbuf[b]
        wbf = wbuf[e]

        def ebody(ed, carry):
            wl = plsc.load_gather(wbf, [jnp.full((16,), ed, jnp.int32)])
            for k in range(D // 16):
                sl = gb[ed, pl.ds(k * 16, 16)]
                gb[ed, pl.ds(k * 16, 16)] = sl * wl
            return carry

        lax.fori_loop(0, CH, ebody, 0, unroll=8)

    def slot(i, j):
        e = j % NI
        b = j % NB
        e2 = (j + 2) % NI
        e3 = (j + 3) % NI
        b2 = (j + 2) % NB

        @pl.when(i + 3 < NCHT)
        def _():
            start_idx(i + 3, e3)

        pltpu.make_async_copy(x_hbm.at[colbuf[e]], gbuf[b], gsem[b]).wait()
        pltpu.make_async_copy(w_hbm.at[pl.ds(0, CH)], wbuf[e], wsem[e]).wait()
        scale(b, e)

        @pl.when(i + 2 < NCHT)
        def _():
            @pl.when(i >= 1)
            def _():
                wait_scatter(b2)

            start_gather(i + 2, e2, b2)

        pltpu.make_async_copy(row_hbm.at[pl.ds(0, CH)], rowbuf[e],
                              rowsem[e]).wait()
        pltpu.async_copy(gbuf[b], acc.at[rowbuf[e]], ssem[b], add=True)

    def lbody(it, carry):
        for j in range(NI):
            slot(it * NI + j, j)
        return carry

    lax.fori_loop(0, NCHT // NI, lbody, 0)
    for b in range(NB):
        wait_scatter(b)
    plsc.subcore_barrier()

    for j in range(RP // CH):
        pltpu.sync_copy(acc.at[pl.ds(r0 + j * CH, CH)], g0.at[pl.ds(0, CH)])
        pltpu.sync_copy(g0.at[pl.ds(0, CH)],
                        out_hbm.at[pl.ds(c * NP + r0 + j * CH, CH)])
    pltpu.sync_copy(acc.at[pl.ds(r0 + (RP // CH) * CH, rem)],
                    g0.at[pl.ds(0, rem)])
    pltpu.sync_copy(g0.at[pl.ds(0, rem)],
                    out_hbm.at[pl.ds(c * NP + r0 + (RP // CH) * CH, rem)])


def _agg(x_all, row, col, w_all):
    mesh = plsc.VectorSubcoreMesh(core_axis_name="c", subcore_axis_name="s")
    f = functools.partial(
        pl.kernel,
        out_type=jax.ShapeDtypeStruct((2 * NP, D), jnp.float32),
        mesh=mesh,
        compiler_params=pltpu.CompilerParams(needs_layout_passes=False),
        scratch_types=(
            [pltpu.VMEM_SHARED((NP, D), jnp.float32)]
            + [pltpu.VMEM((CH,), jnp.int32) for _ in range(NI)]
            + [pltpu.VMEM((CH,), jnp.int32) for _ in range(NI)]
            + [pltpu.VMEM((CH,), jnp.float32) for _ in range(NI)]
            + [pltpu.VMEM((CH, D), jnp.float32) for _ in range(NB)]
            + [pltpu.SemaphoreType.DMA for _ in range(3 * NI + 2 * NB)]
        ),
    )(_agg_body)
    return f(x_all, row, col, w_all)


def kernel(mean, var, edge_index, edge_weight0, edge_weight1,
           W_mean, b_mean, W_var, b_var):
    xs = _dense(mean, var, W_mean, b_mean.reshape(1, D),
                W_var, b_var.reshape(1, D))
    x_all = xs.reshape(2 * N, D)
    pad = EPAD - E
    row = jnp.pad(edge_index[0], (0, pad))
    col = jnp.pad(edge_index[1], (0, pad))
    w_all = jnp.concatenate([
        jnp.pad(edge_weight0, (0, pad)),
        jnp.pad(edge_weight1, (0, pad)),
    ])
    out = _agg(x_all, row, col, w_all)
    return out[:N], out[NP:NP + N]


# speedup vs baseline: 6.8299x; 1.0609x over previous
import functools

import jax
import jax.numpy as jnp
from jax import lax
from jax.experimental import pallas as pl
from jax.experimental.pallas import tpu as pltpu
from jax.experimental.pallas import tpu_sc as plsc

N = 10000
E = 320000
D = 128

NSUB = 16
CH = 112
NCHT = 180
EPAD = NSUB * NCHT * CH
NP = 10112
RP = NP // NSUB
NB = 3
NI = 6


_BLK = 2000


def _dense_body(mean_ref, var_ref, wm_ref, bm_ref, wv_ref, bv_ref, out_ref):
    m = jnp.dot(mean_ref[...], wm_ref[...], preferred_element_type=jnp.float32)
    m = m + bm_ref[...]
    m = jnp.where(m > 0, m, jnp.exp(jnp.minimum(m, 0.0)) - 1.0)
    v = jnp.dot(var_ref[...], wv_ref[...], preferred_element_type=jnp.float32)
    v = v + bv_ref[...]
    v = jnp.maximum(v, 0.0) + 1e-6
    att = jnp.exp(-v)
    out_ref[0] = m * att
    out_ref[1] = v * att * att


def _dense(mean, var, wm, bm, wv, bv):
    grid = (N // _BLK,)
    return pl.pallas_call(
        _dense_body,
        grid=grid,
        in_specs=[
            pl.BlockSpec((_BLK, D), lambda i: (i, 0)),
            pl.BlockSpec((_BLK, D), lambda i: (i, 0)),
            pl.BlockSpec((D, D), lambda i: (0, 0)),
            pl.BlockSpec((1, D), lambda i: (0, 0)),
            pl.BlockSpec((D, D), lambda i: (0, 0)),
            pl.BlockSpec((1, D), lambda i: (0, 0)),
        ],
        out_specs=pl.BlockSpec((2, _BLK, D), lambda i: (0, i, 0)),
        out_shape=jax.ShapeDtypeStruct((2, N, D), jnp.float32),
    )(mean, var, wm, bm, wv, bv)


def _agg_body(x_hbm, row_hbm, col_hbm, w_hbm, out_hbm, acc, *scr):
    colbuf = scr[0:NI]
    rowbuf = scr[NI:2 * NI]
    wbuf = scr[2 * NI:3 * NI]
    gbuf = scr[3 * NI:3 * NI + NB]
    o = 3 * NI + NB
    colsem = scr[o:o + NI]
    rowsem = scr[o + NI:o + 2 * NI]
    wsem = scr[o + 2 * NI:o + 3 * NI]
    gsem = scr[o + 3 * NI:o + 3 * NI + NB]
    ssem = scr[o + 3 * NI + NB:o + 3 * NI + 2 * NB]

    c = lax.axis_index("c")
    s = lax.axis_index("s")
    cN = c * N
    cE = c * EPAD
    z16 = jnp.zeros((16,), jnp.float32)
    t0 = s * NCHT
    r0 = s * RP
    g0 = gbuf[0]

    def start_idx(i, e):
        off = (t0 + i) * CH
        pltpu.async_copy(col_hbm.at[pl.ds(off, CH)], colbuf[e], colsem[e])
        pltpu.async_copy(row_hbm.at[pl.ds(off, CH)], rowbuf[e], rowsem[e])
        pltpu.async_copy(w_hbm.at[pl.ds(cE + off, CH)], wbuf[e], wsem[e])

    def start_gather(i, e, b):
        off = (t0 + i) * CH
        pltpu.make_async_copy(col_hbm.at[pl.ds(off, CH)], colbuf[e],
                              colsem[e]).wait()
        for k in range(CH // 16):
            colbuf[e][pl.ds(k * 16, 16)] = colbuf[e][pl.ds(k * 16, 16)] + cN
        pltpu.async_copy(x_hbm.at[colbuf[e]], gbuf[b], gsem[b])

    def wait_scatter(b):
        pltpu.make_async_copy(gbuf[b], acc.at[rowbuf[0]], ssem[b]).wait()

    for i in range(3):
        start_idx(i, i)

    def zbody(r, carry):
        for k in range(D // 16):
            g0[r, pl.ds(k * 16, 16)] = z16
        return carry

    lax.fori_loop(0, CH, zbody, 0)
    for j in range(RP // CH):
        pltpu.sync_copy(g0.at[pl.ds(0, CH)], acc.at[pl.ds(r0 + j * CH, CH)])
    rem = RP - (RP // CH) * CH
    pltpu.sync_copy(g0.at[pl.ds(0, rem)],
                    acc.at[pl.ds(r0 + (RP // CH) * CH, rem)])

    start_gather(0, 0, 0)
    start_gather(1, 1, 1)
    plsc.subcore_barrier()

    def scale(b, e):
        gb = gbuf[b]
        wbf = wbuf[e]

        def bbody(b16, carry):
            wv = wbf[pl.ds(b16 * 16, 16)]
            for l in range(16):
                ed = b16 * 16 + l
                wl = wv[l]
                for k in range(D // 16):
                    sl = gb[ed, pl.ds(k * 16, 16)]
                    gb[ed, pl.ds(k * 16, 16)] = sl * wl
            return carry

        lax.fori_loop(0, CH // 16, bbody, 0)

    def slot(i, j):
        e = j % NI
        b = j % NB
        e2 = (j + 2) % NI
        e3 = (j + 3) % NI
        b2 = (j + 2) % NB

        @pl.when(i + 3 < NCHT)
        def _():
            start_idx(i + 3, e3)

        pltpu.make_async_copy(x_hbm.at[colbuf[e]], gbuf[b], gsem[b]).wait()
        pltpu.make_async_copy(w_hbm.at[pl.ds(0, CH)], wbuf[e], wsem[e]).wait()
        scale(b, e)

        @pl.when(i + 2 < NCHT)
        def _():
            @pl.when(i >= 1)
            def _():
                wait_scatter(b2)

            start_gather(i + 2, e2, b2)

        pltpu.make_async_copy(row_hbm.at[pl.ds(0, CH)], rowbuf[e],
                              rowsem[e]).wait()
        pltpu.async_copy(gbuf[b], acc.at[rowbuf[e]], ssem[b], add=True)

    def lbody(it, carry):
        for j in range(NI):
            slot(it * NI + j, j)
        return carry

    lax.fori_loop(0, NCHT // NI, lbody, 0)
    for b in range(NB):
        wait_scatter(b)
    plsc.subcore_barrier()

    for j in range(RP // CH):
        pltpu.sync_copy(acc.at[pl.ds(r0 + j * CH, CH)], g0.at[pl.ds(0, CH)])
        pltpu.sync_copy(g0.at[pl.ds(0, CH)],
                        out_hbm.at[pl.ds(c * NP + r0 + j * CH, CH)])
    pltpu.sync_copy(acc.at[pl.ds(r0 + (RP // CH) * CH, rem)],
                    g0.at[pl.ds(0, rem)])
    pltpu.sync_copy(g0.at[pl.ds(0, rem)],
                    out_hbm.at[pl.ds(c * NP + r0 + (RP // CH) * CH, rem)])


def _agg(x_all, row, col, w_all):
    mesh = plsc.VectorSubcoreMesh(core_axis_name="c", subcore_axis_name="s")
    f = functools.partial(
        pl.kernel,
        out_type=jax.ShapeDtypeStruct((2 * NP, D), jnp.float32),
        mesh=mesh,
        compiler_params=pltpu.CompilerParams(needs_layout_passes=False),
        scratch_types=(
            [pltpu.VMEM_SHARED((NP, D), jnp.float32)]
            + [pltpu.VMEM((CH,), jnp.int32) for _ in range(NI)]
            + [pltpu.VMEM((CH,), jnp.int32) for _ in range(NI)]
            + [pltpu.VMEM((CH,), jnp.float32) for _ in range(NI)]
            + [pltpu.VMEM((CH, D), jnp.float32) for _ in range(NB)]
            + [pltpu.SemaphoreType.DMA for _ in range(3 * NI + 2 * NB)]
        ),
    )(_agg_body)
    return f(x_all, row, col, w_all)


def kernel(mean, var, edge_index, edge_weight0, edge_weight1,
           W_mean, b_mean, W_var, b_var):
    xs = _dense(mean, var, W_mean, b_mean.reshape(1, D),
                W_var, b_var.reshape(1, D))
    x_all = xs.reshape(2 * N, D)
    pad = EPAD - E
    row = jnp.pad(edge_index[0], (0, pad))
    col = jnp.pad(edge_index[1], (0, pad))
    w_all = jnp.concatenate([
        jnp.pad(edge_weight0, (0, pad)),
        jnp.pad(edge_weight1, (0, pad)),
    ])
    out = _agg(x_all, row, col, w_all)
    return out[:N], out[NP:NP + N]
